# baseline (speedup 1.0000x reference)
import functools

import jax
import jax.numpy as jnp
from jax import lax
from jax.experimental import pallas as pl
from jax.experimental.pallas import tpu as pltpu
from jax.experimental.pallas import tpu_sc as plsc

DIM = 32
B_TOTAL = 16384 * 50
NUM_WORKERS = 32
B_PER_W = B_TOTAL // NUM_WORKERS
CHUNK = 1280
N_CHUNKS = B_PER_W // CHUNK
N_STREAMS = 4
SUB = CHUNK // N_STREAMS


def _emb_body(w_hbm, idx_hbm, out_hbm,
              idx0, idx1, rows0, rows1,
              s_i0, s_i1, s_g0, s_g1, s_o0, s_o1):
    idx_v = [idx0, idx1]
    rows_v = [rows0, rows1]
    s_i = [s_i0, s_i1]
    s_g = [s_g0, s_g1]
    s_o = [s_o0, s_o1]

    wid = lax.axis_index("s") * 2 + lax.axis_index("c")
    base = wid * B_PER_W

    def idx_copy(chunk, b):
        return pltpu.make_async_copy(
            idx_hbm.at[pl.ds(base + chunk * CHUNK, CHUNK)], idx_v[b], s_i[b])

    def _gather_descs(b):
        return [pltpu.make_async_copy(
                    w_hbm.at[idx_v[b].at[pl.ds(j * SUB, SUB)]],
                    rows_v[b].at[pl.ds(j * SUB, SUB)], s_g[b])
                for j in range(N_STREAMS)]

    class gather:
        def __init__(self, b):
            self.b = b

        def start(self):
            for d in _gather_descs(self.b):
                d.start()

        def wait(self):
            for d in _gather_descs(self.b):
                d.wait()

    def writeback(chunk, b):
        return pltpu.make_async_copy(
            rows_v[b], out_hbm.at[pl.ds(base + chunk * CHUNK, CHUNK)], s_o[b])

    idx_copy(0, 0).start()
    idx_copy(1, 1).start()
    idx_copy(0, 0).wait()
    gather(0).start()

    def outer(g, carry):
        idx_copy(g + 1, 1).wait()

        @pl.when(g >= 1)
        def _():
            writeback(g - 1, 1).wait()

        gather(1).start()
        gather(0).wait()
        writeback(g, 0).start()

        @pl.when(g < N_CHUNKS - 2)
        def _():
            idx_copy(g + 2, 0).start()

        @pl.when(g < N_CHUNKS - 2)
        def _():
            idx_copy(g + 2, 0).wait()
            writeback(g, 0).wait()
            gather(0).start()

        gather(1).wait()
        writeback(g + 1, 1).start()

        @pl.when(g < N_CHUNKS - 2)
        def _():
            idx_copy(g + 3, 1).start()

        return carry

    lax.fori_loop(0, N_CHUNKS // 2, lambda j, c: outer(j * 2, c), 0,
                  unroll=False)

    writeback(N_CHUNKS - 2, 0).wait()
    writeback(N_CHUNKS - 1, 1).wait()


@jax.jit
def _embed(W, idx_flat):
    mesh = plsc.VectorSubcoreMesh(core_axis_name="c", subcore_axis_name="s")
    f = functools.partial(
        pl.kernel,
        mesh=mesh,
        out_type=jax.ShapeDtypeStruct((B_TOTAL, DIM), jnp.float32),
        scratch_types=[
            pltpu.VMEM((CHUNK,), jnp.int32),
            pltpu.VMEM((CHUNK,), jnp.int32),
            pltpu.VMEM((CHUNK, DIM), jnp.float32),
            pltpu.VMEM((CHUNK, DIM), jnp.float32),
            pltpu.SemaphoreType.DMA,
            pltpu.SemaphoreType.DMA,
            pltpu.SemaphoreType.DMA,
            pltpu.SemaphoreType.DMA,
            pltpu.SemaphoreType.DMA,
            pltpu.SemaphoreType.DMA,
        ],
        compiler_params=pltpu.CompilerParams(use_tc_tiling_on_sc=False),
    )(_emb_body)
    return f(W, idx_flat)


_TT = 512


def _tr_body(x_ref, o_ref):
    o_ref[0] = jnp.transpose(x_ref[0], (1, 0))


def _transpose_tc(x3):
    n_s, n_t, _ = x3.shape
    return pl.pallas_call(
        _tr_body,
        grid=(n_s, n_t // _TT),
        in_specs=[pl.BlockSpec((1, _TT, DIM), lambda s, j: (s, j, 0))],
        out_specs=pl.BlockSpec((1, DIM, _TT), lambda s, j: (s, 0, j)),
        out_shape=jax.ShapeDtypeStruct((n_s, DIM, n_t), jnp.float32),
    )(x3)


def kernel(token_ids, W):
    n_t, n_s = token_ids.shape
    idx_s = jnp.transpose(token_ids).reshape(-1).astype(jnp.int32)
    x = _embed(W, idx_s)
    y = _transpose_tc(x.reshape(n_s, n_t, DIM))
    return jnp.transpose(y, (2, 0, 1))


# speedup vs baseline: 1.7682x; 1.7682x over previous
import functools

import jax
import jax.numpy as jnp
from jax import lax
from jax.experimental import pallas as pl
from jax.experimental.pallas import tpu as pltpu
from jax.experimental.pallas import tpu_sc as plsc

DIM = 32
B_TOTAL = 16384 * 50
NUM_WORKERS = 32
B_PER_W = B_TOTAL // NUM_WORKERS
CHUNK = 1280
N_CHUNKS = B_PER_W // CHUNK
N_STREAMS = 4
SUB = CHUNK // N_STREAMS


def _emb_body(w_hbm, idx_hbm, out_hbm,
              idx0, idx1, rows0, rows1,
              s_i0, s_i1, s_g0, s_g1, s_o0, s_o1):
    idx_v = [idx0, idx1]
    rows_v = [rows0, rows1]
    s_i = [s_i0, s_i1]
    s_g = [s_g0, s_g1]
    s_o = [s_o0, s_o1]

    wid = lax.axis_index("s") * 2 + lax.axis_index("c")
    base = wid * B_PER_W

    def idx_copy(chunk, b):
        return pltpu.make_async_copy(
            idx_hbm.at[pl.ds(base + chunk * CHUNK, CHUNK)], idx_v[b], s_i[b])

    def _gather_descs(b):
        return [pltpu.make_async_copy(
                    w_hbm.at[idx_v[b].at[pl.ds(j * SUB, SUB)]],
                    rows_v[b].at[pl.ds(j * SUB, SUB)], s_g[b])
                for j in range(N_STREAMS)]

    class gather:
        def __init__(self, b):
            self.b = b

        def start(self):
            for d in _gather_descs(self.b):
                d.start()

        def wait(self):
            for d in _gather_descs(self.b):
                d.wait()

    def writeback(chunk, b):
        return pltpu.make_async_copy(
            rows_v[b], out_hbm.at[pl.ds(base + chunk * CHUNK, CHUNK)], s_o[b])

    idx_copy(0, 0).start()
    idx_copy(1, 1).start()
    idx_copy(0, 0).wait()
    gather(0).start()

    def outer(g, carry):
        idx_copy(g + 1, 1).wait()

        @pl.when(g >= 1)
        def _():
            writeback(g - 1, 1).wait()

        gather(1).start()
        gather(0).wait()
        writeback(g, 0).start()

        @pl.when(g < N_CHUNKS - 2)
        def _():
            idx_copy(g + 2, 0).start()

        @pl.when(g < N_CHUNKS - 2)
        def _():
            idx_copy(g + 2, 0).wait()
            writeback(g, 0).wait()
            gather(0).start()

        gather(1).wait()
        writeback(g + 1, 1).start()

        @pl.when(g < N_CHUNKS - 2)
        def _():
            idx_copy(g + 3, 1).start()

        return carry

    lax.fori_loop(0, N_CHUNKS // 2, lambda j, c: outer(j * 2, c), 0,
                  unroll=False)

    writeback(N_CHUNKS - 2, 0).wait()
    writeback(N_CHUNKS - 1, 1).wait()


@jax.jit
def _embed(W, idx_flat):
    mesh = plsc.VectorSubcoreMesh(core_axis_name="c", subcore_axis_name="s")
    f = functools.partial(
        pl.kernel,
        mesh=mesh,
        out_type=jax.ShapeDtypeStruct((B_TOTAL, DIM), jnp.float32),
        scratch_types=[
            pltpu.VMEM((CHUNK,), jnp.int32),
            pltpu.VMEM((CHUNK,), jnp.int32),
            pltpu.VMEM((CHUNK, DIM), jnp.float32),
            pltpu.VMEM((CHUNK, DIM), jnp.float32),
            pltpu.SemaphoreType.DMA,
            pltpu.SemaphoreType.DMA,
            pltpu.SemaphoreType.DMA,
            pltpu.SemaphoreType.DMA,
            pltpu.SemaphoreType.DMA,
            pltpu.SemaphoreType.DMA,
        ],
        compiler_params=pltpu.CompilerParams(use_tc_tiling_on_sc=False),
    )(_emb_body)
    return f(W, idx_flat)


def _tr_body(x_ref, o_ref):
    o_ref[0] = jnp.transpose(x_ref[0], (1, 0))


def _transpose_tc(x3):
    n_s, n_t, _ = x3.shape
    return pl.pallas_call(
        _tr_body,
        grid=(n_s,),
        in_specs=[pl.BlockSpec((1, n_t, DIM), lambda s: (s, 0, 0))],
        out_specs=pl.BlockSpec((1, DIM, n_t), lambda s: (s, 0, 0)),
        out_shape=jax.ShapeDtypeStruct((n_s, DIM, n_t), jnp.float32),
    )(x3)


def kernel(token_ids, W):
    n_t, n_s = token_ids.shape
    idx_s = jnp.transpose(token_ids).reshape(-1).astype(jnp.int32)
    x = _embed(W, idx_s)
    y = _transpose_tc(x.reshape(n_s, n_t, DIM))
    return jnp.transpose(y, (2, 0, 1))


# speedup vs baseline: 2.1483x; 1.2149x over previous
import functools

import jax
import jax.numpy as jnp
from jax import lax
from jax.experimental import pallas as pl
from jax.experimental.pallas import tpu as pltpu
from jax.experimental.pallas import tpu_sc as plsc

DIM = 32
B_TOTAL = 16384 * 50
NUM_WORKERS = 32
B_PER_W = B_TOTAL // NUM_WORKERS
CHUNK = 1280
N_CHUNKS = B_PER_W // CHUNK
N_STREAMS = 4
SUB = CHUNK // N_STREAMS


def _emb_body(w_hbm, idx_hbm, out_hbm,
              idx0, idx1, rows0, rows1,
              s_i0, s_i1, s_g0, s_g1, s_o0, s_o1):
    idx_v = [idx0, idx1]
    rows_v = [rows0, rows1]
    s_i = [s_i0, s_i1]
    s_g = [s_g0, s_g1]
    s_o = [s_o0, s_o1]

    wid = lax.axis_index("s") * 2 + lax.axis_index("c")
    base = wid * B_PER_W

    def idx_copy(chunk, b):
        return pltpu.make_async_copy(
            idx_hbm.at[pl.ds(base + chunk * CHUNK, CHUNK)], idx_v[b], s_i[b])

    def _gather_descs(b):
        return [pltpu.make_async_copy(
                    w_hbm.at[idx_v[b].at[pl.ds(j * SUB, SUB)]],
                    rows_v[b].at[pl.ds(j * SUB, SUB)], s_g[b])
                for j in range(N_STREAMS)]

    class gather:
        def __init__(self, b):
            self.b = b

        def start(self):
            for d in _gather_descs(self.b):
                d.start()

        def wait(self):
            for d in _gather_descs(self.b):
                d.wait()

    def writeback(chunk, b):
        return pltpu.make_async_copy(
            rows_v[b],
            out_hbm.at[pl.ds(base + chunk * CHUNK, CHUNK), pl.ds(0, DIM)],
            s_o[b])

    idx_copy(0, 0).start()
    idx_copy(1, 1).start()
    idx_copy(0, 0).wait()
    gather(0).start()

    def outer(g, carry):
        idx_copy(g + 1, 1).wait()

        @pl.when(g >= 1)
        def _():
            writeback(g - 1, 1).wait()

        gather(1).start()
        gather(0).wait()
        writeback(g, 0).start()

        @pl.when(g < N_CHUNKS - 2)
        def _():
            idx_copy(g + 2, 0).start()

        @pl.when(g < N_CHUNKS - 2)
        def _():
            idx_copy(g + 2, 0).wait()
            writeback(g, 0).wait()
            gather(0).start()

        gather(1).wait()
        writeback(g + 1, 1).start()

        @pl.when(g < N_CHUNKS - 2)
        def _():
            idx_copy(g + 3, 1).start()

        return carry

    lax.fori_loop(0, N_CHUNKS // 2, lambda j, c: outer(j * 2, c), 0,
                  unroll=False)

    writeback(N_CHUNKS - 2, 0).wait()
    writeback(N_CHUNKS - 1, 1).wait()


@jax.jit
def _embed(W, idx_flat):
    mesh = plsc.VectorSubcoreMesh(core_axis_name="c", subcore_axis_name="s")
    f = functools.partial(
        pl.kernel,
        mesh=mesh,
        out_type=jax.ShapeDtypeStruct((B_TOTAL, 128), jnp.float32),
        scratch_types=[
            pltpu.VMEM((CHUNK,), jnp.int32),
            pltpu.VMEM((CHUNK,), jnp.int32),
            pltpu.VMEM((CHUNK, DIM), jnp.float32),
            pltpu.VMEM((CHUNK, DIM), jnp.float32),
            pltpu.SemaphoreType.DMA,
            pltpu.SemaphoreType.DMA,
            pltpu.SemaphoreType.DMA,
            pltpu.SemaphoreType.DMA,
            pltpu.SemaphoreType.DMA,
            pltpu.SemaphoreType.DMA,
        ],
        compiler_params=pltpu.CompilerParams(use_tc_tiling_on_sc=False),
    )(_emb_body)
    return f(W, idx_flat)


def _tr_body(x_ref, o_ref):
    o_ref[0] = jnp.transpose(x_ref[0][:, 0:DIM], (1, 0))


_BT = 4096


def _transpose_tc(x3):
    n_s, n_t, _ = x3.shape
    return pl.pallas_call(
        _tr_body,
        grid=(n_s, n_t // _BT),
        in_specs=[pl.BlockSpec((1, _BT, 128), lambda s, j: (s, j, 0))],
        out_specs=pl.BlockSpec((1, DIM, _BT), lambda s, j: (s, 0, j)),
        out_shape=jax.ShapeDtypeStruct((n_s, DIM, n_t), jnp.float32),
    )(x3)


def kernel(token_ids, W):
    n_t, n_s = token_ids.shape
    idx_s = jnp.transpose(token_ids).reshape(-1).astype(jnp.int32)
    x = _embed(W, idx_s)
    y = _transpose_tc(x.reshape(n_s, n_t, 128))
    return jnp.transpose(y, (2, 0, 1))
